# Initial kernel scaffold; baseline (speedup 1.0000x reference)
import jax
import jax.numpy as jnp
from jax.experimental import pallas as pl


def kernel(text_ids, table):
    raise NotImplementedError("write your pallas kernel here")


# speedup vs baseline: 2.6458x; 2.6458x over previous
import functools

import jax
import jax.numpy as jnp
from jax import lax
from jax.experimental import pallas as pl
from jax.experimental.pallas import tpu as pltpu
from jax.experimental.pallas import tpu_sc as plsc

_B = 16384
_S = 50
_D = 64
_NW = 32
_RPW = _B // _NW
_R = 16
_NCHUNK = _RPW // _R
_RS = _R * _S


@functools.partial(
    pl.kernel,
    mesh=plsc.VectorSubcoreMesh(core_axis_name="c", subcore_axis_name="s"),
    out_type=jax.ShapeDtypeStruct((_B, _D), jnp.float32),
    compiler_params=pltpu.CompilerParams(use_tc_tiling_on_sc=False),
    scratch_types=[
        pltpu.VMEM((_RS,), jnp.int32),
        pltpu.VMEM((_RS,), jnp.int32),
        pltpu.VMEM((_RS, _D), jnp.float32),
        pltpu.VMEM((_RS, _D), jnp.float32),
        pltpu.VMEM((_RS,), jnp.int32),
        pltpu.VMEM((_R, _D), jnp.float32),
        pltpu.SemaphoreType.DMA,
        pltpu.SemaphoreType.DMA,
    ],
)
def _pool_kernel(idsf, idst, table, out,
                 idx0, idx1, rows0, rows1, idxt, outb, sem0, sem1):
    cid = lax.axis_index("c")
    sid = lax.axis_index("s")
    wid = sid * 2 + cid
    wbase = wid * _RPW

    ones = jnp.ones((16,), jnp.float32)
    zeros = jnp.zeros((16,), jnp.float32)

    def idx_copy(ci, idxref):
        pltpu.sync_copy(idsf.at[pl.ds((wbase + ci * _R) * _S, _RS)], idxref)

    def gather(idxref, rowsref, sem):
        return pltpu.make_async_copy(table.at[idxref], rowsref, sem)

    def process(ci, rowsref):
        base = wbase + ci * _R

        pltpu.sync_copy(idst.at[pl.ds(base * _S, _RS)], idxt)

        def cnt_body(l, cv):
            return cv + jnp.where(idxt[pl.ds(l * 16, 16)] != 0, ones, zeros)

        cntv = lax.fori_loop(0, _S, cnt_body, zeros)
        denv = jnp.maximum(cntv, 1.0)

        def row_body(r, carry):
            o = r * _S
            denom = lax.gather(
                denv, jnp.full((16, 1), r, jnp.int32),
                lax.GatherDimensionNumbers(
                    offset_dims=(), collapsed_slice_dims=(0,),
                    start_index_map=(0,)),
                (1,), mode=lax.GatherScatterMode.PROMISE_IN_BOUNDS)

            def l_body(lo, accs):
                a0, a1, a2, a3 = accs
                for u in range(5):
                    ro = o + lo * 5 + u
                    a0 = a0 + rowsref[ro, pl.ds(0, 16)]
                    a1 = a1 + rowsref[ro, pl.ds(16, 16)]
                    a2 = a2 + rowsref[ro, pl.ds(32, 16)]
                    a3 = a3 + rowsref[ro, pl.ds(48, 16)]
                return (a0, a1, a2, a3)

            a0, a1, a2, a3 = lax.fori_loop(0, _S // 5, l_body,
                                           (zeros, zeros, zeros, zeros))
            outb[r, pl.ds(0, 16)] = a0 / denom
            outb[r, pl.ds(16, 16)] = a1 / denom
            outb[r, pl.ds(32, 16)] = a2 / denom
            outb[r, pl.ds(48, 16)] = a3 / denom
            return carry

        lax.fori_loop(0, _R, row_body, 0)
        pltpu.sync_copy(outb, out.at[pl.ds(base, _R)])

    idx_copy(0, idx0)
    gather(idx0, rows0, sem0).start()

    def outer(i, carry):
        c0 = 2 * i
        c1 = 2 * i + 1
        idx_copy(c1, idx1)
        gather(idx1, rows1, sem1).start()
        gather(idx0, rows0, sem0).wait()
        process(c0, rows0)
        cnext = jnp.minimum(c0 + 2, _NCHUNK - 1)
        idx_copy(cnext, idx0)
        gather(idx0, rows0, sem0).start()
        gather(idx1, rows1, sem1).wait()
        process(c1, rows1)
        return carry

    lax.fori_loop(0, _NCHUNK // 2, outer, 0)
    gather(idx0, rows0, sem0).wait()


def kernel(text_ids, table):
    ids_flat = text_ids.reshape(-1)
    ids_t = (text_ids.reshape(_B // _R, _R, _S)
             .transpose(0, 2, 1)
             .reshape(-1))
    return _pool_kernel(ids_flat, ids_t, table)


# speedup vs baseline: 2.6520x; 1.0024x over previous
import functools

import jax
import jax.numpy as jnp
from jax import lax
from jax.experimental import pallas as pl
from jax.experimental.pallas import tpu as pltpu
from jax.experimental.pallas import tpu_sc as plsc

_B = 16384
_S = 50
_D = 64
_NW = 32
_RPW = _B // _NW
_R = 16
_NCHUNK = _RPW // _R
_RS = _R * _S


@functools.partial(
    pl.kernel,
    mesh=plsc.VectorSubcoreMesh(core_axis_name="c", subcore_axis_name="s"),
    out_type=jax.ShapeDtypeStruct((_B, _D), jnp.float32),
    compiler_params=pltpu.CompilerParams(use_tc_tiling_on_sc=False),
    scratch_types=[
        pltpu.VMEM((_RS,), jnp.int32),
        pltpu.VMEM((_RS,), jnp.int32),
        pltpu.VMEM((_RS, _D), jnp.float32),
        pltpu.VMEM((_RS, _D), jnp.float32),
        pltpu.VMEM((_RS,), jnp.int32),
        pltpu.VMEM((_R, _D), jnp.float32),
        pltpu.SemaphoreType.DMA,
        pltpu.SemaphoreType.DMA,
    ],
)
def _pool_kernel(idsf, idst, table, out,
                 idx0, idx1, rows0, rows1, idxt, outb, sem0, sem1):
    cid = lax.axis_index("c")
    sid = lax.axis_index("s")
    wid = sid * 2 + cid
    wbase = wid * _RPW

    ones = jnp.ones((16,), jnp.float32)
    zeros = jnp.zeros((16,), jnp.float32)

    def idx_copy(ci, idxref):
        pltpu.sync_copy(idsf.at[pl.ds((wbase + ci * _R) * _S, _RS)], idxref)

    def gather(idxref, rowsref, sem):
        return pltpu.make_async_copy(table.at[idxref], rowsref, sem)

    def process(ci, rowsref):
        base = wbase + ci * _R

        pltpu.sync_copy(idst.at[pl.ds(base * _S, _RS)], idxt)

        def cnt_body(l, cv):
            return cv + jnp.where(idxt[pl.ds(l * 16, 16)] != 0, ones, zeros)

        cntv = lax.fori_loop(0, _S, cnt_body, zeros)
        denv = jnp.maximum(cntv, 1.0)

        for g in range(_R // 4):
            @plsc.parallel_loop(0, _S, carry=(zeros,) * 16)
            def acc_body(l, accs, g=g):
                res = []
                for rr in range(4):
                    o = (g * 4 + rr) * _S + l
                    for c in range(4):
                        res.append(accs[rr * 4 + c]
                                   + rowsref[o, pl.ds(c * 16, 16)])
                return tuple(res)

            accs = acc_body
            for rr in range(4):
                r = g * 4 + rr
                denom = lax.gather(
                    denv, jnp.full((16, 1), r, jnp.int32),
                    lax.GatherDimensionNumbers(
                        offset_dims=(), collapsed_slice_dims=(0,),
                        start_index_map=(0,)),
                    (1,), mode=lax.GatherScatterMode.PROMISE_IN_BOUNDS)
                for c in range(4):
                    outb[r, pl.ds(c * 16, 16)] = accs[rr * 4 + c] / denom

        pltpu.sync_copy(outb, out.at[pl.ds(base, _R)])

    idx_copy(0, idx0)
    gather(idx0, rows0, sem0).start()

    def outer(i, carry):
        c0 = 2 * i
        c1 = 2 * i + 1
        idx_copy(c1, idx1)
        gather(idx1, rows1, sem1).start()
        gather(idx0, rows0, sem0).wait()
        process(c0, rows0)
        cnext = jnp.minimum(c0 + 2, _NCHUNK - 1)
        idx_copy(cnext, idx0)
        gather(idx0, rows0, sem0).start()
        gather(idx1, rows1, sem1).wait()
        process(c1, rows1)
        return carry

    lax.fori_loop(0, _NCHUNK // 2, outer, 0)
    gather(idx0, rows0, sem0).wait()


def kernel(text_ids, table):
    ids_flat = text_ids.reshape(-1)
    ids_t = (text_ids.reshape(_B // _R, _R, _S)
             .transpose(0, 2, 1)
             .reshape(-1))
    return _pool_kernel(ids_flat, ids_t, table)


# speedup vs baseline: 2.6582x; 1.0023x over previous
import functools

import jax
import jax.numpy as jnp
from jax import lax
from jax.experimental import pallas as pl
from jax.experimental.pallas import tpu as pltpu
from jax.experimental.pallas import tpu_sc as plsc

_B = 16384
_S = 50
_D = 64
_NW = 32
_RPW = _B // _NW
_R = 16
_NCHUNK = _RPW // _R
_RS = _R * _S


@functools.partial(
    pl.kernel,
    mesh=plsc.VectorSubcoreMesh(core_axis_name="c", subcore_axis_name="s"),
    out_type=jax.ShapeDtypeStruct((_B, _D), jnp.float32),
    compiler_params=pltpu.CompilerParams(use_tc_tiling_on_sc=False),
    scratch_types=[
        pltpu.VMEM((_RS,), jnp.int32),
        pltpu.VMEM((_RS,), jnp.int32),
        pltpu.VMEM((_RS, _D), jnp.float32),
        pltpu.VMEM((_RS, _D), jnp.float32),
        pltpu.VMEM((_RS,), jnp.int32),
        pltpu.VMEM((_R, _D), jnp.float32),
        pltpu.SemaphoreType.DMA,
        pltpu.SemaphoreType.DMA,
    ],
)
def _pool_kernel(idsf, idst, table, out,
                 idx0, idx1, rows0, rows1, idxt, outb, sem0, sem1):
    cid = lax.axis_index("c")
    sid = lax.axis_index("s")
    wid = sid * 2 + cid
    wbase = wid * _RPW

    ones = jnp.ones((16,), jnp.float32)
    zeros = jnp.zeros((16,), jnp.float32)

    def idx_copy(ci, idxref):
        pltpu.sync_copy(idsf.at[pl.ds((wbase + ci * _R) * _S, _RS)], idxref)

    _H = _RS // 2

    def gather_start(idxref, rowsref, sem):
        pltpu.make_async_copy(
            table.at[idxref.at[pl.ds(0, _H)]],
            rowsref.at[pl.ds(0, _H)], sem).start()
        pltpu.make_async_copy(
            table.at[idxref.at[pl.ds(_H, _H)]],
            rowsref.at[pl.ds(_H, _H)], sem).start()

    def gather_wait(idxref, rowsref, sem):
        pltpu.make_async_copy(
            table.at[idxref.at[pl.ds(0, _H)]],
            rowsref.at[pl.ds(0, _H)], sem).wait()
        pltpu.make_async_copy(
            table.at[idxref.at[pl.ds(_H, _H)]],
            rowsref.at[pl.ds(_H, _H)], sem).wait()

    def process(ci, rowsref):
        base = wbase + ci * _R

        pltpu.sync_copy(idst.at[pl.ds(base * _S, _RS)], idxt)

        def cnt_body(l, cv):
            return cv + jnp.where(idxt[pl.ds(l * 16, 16)] != 0, ones, zeros)

        cntv = lax.fori_loop(0, _S, cnt_body, zeros)
        denv = jnp.maximum(cntv, 1.0)

        for g in range(_R // 4):
            @plsc.parallel_loop(0, _S, carry=(zeros,) * 16)
            def acc_body(l, accs, g=g):
                res = []
                for rr in range(4):
                    o = (g * 4 + rr) * _S + l
                    for c in range(4):
                        res.append(accs[rr * 4 + c]
                                   + rowsref[o, pl.ds(c * 16, 16)])
                return tuple(res)

            accs = acc_body
            for rr in range(4):
                r = g * 4 + rr
                denom = lax.gather(
                    denv, jnp.full((16, 1), r, jnp.int32),
                    lax.GatherDimensionNumbers(
                        offset_dims=(), collapsed_slice_dims=(0,),
                        start_index_map=(0,)),
                    (1,), mode=lax.GatherScatterMode.PROMISE_IN_BOUNDS)
                for c in range(4):
                    outb[r, pl.ds(c * 16, 16)] = accs[rr * 4 + c] / denom

        pltpu.sync_copy(outb, out.at[pl.ds(base, _R)])

    idx_copy(0, idx0)
    gather_start(idx0, rows0, sem0)

    def outer(i, carry):
        c0 = 2 * i
        c1 = 2 * i + 1
        idx_copy(c1, idx1)
        gather_start(idx1, rows1, sem1)
        gather_wait(idx0, rows0, sem0)
        process(c0, rows0)
        cnext = jnp.minimum(c0 + 2, _NCHUNK - 1)
        idx_copy(cnext, idx0)
        gather_start(idx0, rows0, sem0)
        gather_wait(idx1, rows1, sem1)
        process(c1, rows1)
        return carry

    lax.fori_loop(0, _NCHUNK // 2, outer, 0)
    gather_wait(idx0, rows0, sem0)


def kernel(text_ids, table):
    ids_flat = text_ids.reshape(-1)
    ids_t = (text_ids.reshape(_B // _R, _R, _S)
             .transpose(0, 2, 1)
             .reshape(-1))
    return _pool_kernel(ids_flat, ids_t, table)


# speedup vs baseline: 2.7870x; 1.0485x over previous
import functools

import jax
import jax.numpy as jnp
from jax import lax
from jax.experimental import pallas as pl
from jax.experimental.pallas import tpu as pltpu
from jax.experimental.pallas import tpu_sc as plsc

_B = 16384
_S = 50
_D = 64
_NW = 32
_RPW = _B // _NW
_R = 16
_NCHUNK = _RPW // _R
_RS = _R * _S


@functools.partial(
    pl.kernel,
    mesh=plsc.VectorSubcoreMesh(core_axis_name="c", subcore_axis_name="s"),
    out_type=jax.ShapeDtypeStruct((_B, _D), jnp.float32),
    compiler_params=pltpu.CompilerParams(use_tc_tiling_on_sc=False),
    scratch_types=[
        pltpu.VMEM((_RS,), jnp.int32),
        pltpu.VMEM((_RS,), jnp.int32),
        pltpu.VMEM((_RS, _D), jnp.float32),
        pltpu.VMEM((_RS, _D), jnp.float32),
        pltpu.VMEM((_R, _D), jnp.float32),
        pltpu.VMEM((_R, _D), jnp.float32),
        pltpu.SemaphoreType.DMA,
        pltpu.SemaphoreType.DMA,
        pltpu.SemaphoreType.DMA,
        pltpu.SemaphoreType.DMA,
        pltpu.SemaphoreType.DMA,
        pltpu.SemaphoreType.DMA,
    ],
)
def _pool_kernel(idsm, table, out, idx0, idx1, rows0, rows1, outb0, outb1,
                 semi0, semi1, semg0, semg1, semo0, semo1):
    cid = lax.axis_index("c")
    sid = lax.axis_index("s")
    wid = sid * 2 + cid
    wbase = wid * _RPW

    ones = jnp.ones((16,), jnp.float32)
    zeros = jnp.zeros((16,), jnp.float32)

    def idx_cp(ci, idxref, sem):
        return pltpu.make_async_copy(
            idsm.at[pl.ds((wbase + ci * _R) * _S, _RS)], idxref, sem)

    def gather(idxref, rowsref, sem):
        return pltpu.make_async_copy(table.at[idxref], rowsref, sem)

    def out_cp(ci, outref, sem):
        return pltpu.make_async_copy(
            outref, out.at[pl.ds(wbase + ci * _R, _R)], sem)

    def count(idxref):
        def cnt_body(l, cv):
            return cv + jnp.where(idxref[pl.ds(l * 16, 16)] != 0, ones, zeros)

        cntv = lax.fori_loop(0, _S, cnt_body, zeros)
        return jnp.maximum(cntv, 1.0)

    def process(ci, denv, rowsref, outref, semo):
        for g in range(_R // 4):
            @plsc.parallel_loop(0, _S, carry=(zeros,) * 16)
            def acc_body(l, accs, g=g):
                res = []
                for rr in range(4):
                    o = l * _R + g * 4 + rr
                    for c in range(4):
                        res.append(accs[rr * 4 + c]
                                   + rowsref[o, pl.ds(c * 16, 16)])
                return tuple(res)

            accs = acc_body
            for rr in range(4):
                r = g * 4 + rr
                denom = lax.gather(
                    denv, jnp.full((16, 1), r, jnp.int32),
                    lax.GatherDimensionNumbers(
                        offset_dims=(), collapsed_slice_dims=(0,),
                        start_index_map=(0,)),
                    (1,), mode=lax.GatherScatterMode.PROMISE_IN_BOUNDS)
                for c in range(4):
                    outref[r, pl.ds(c * 16, 16)] = accs[rr * 4 + c] / denom

        out_cp(ci, outref, semo).start()

    idx_cp(0, idx0, semi0).start()
    idx_cp(1, idx1, semi1).start()
    idx_cp(0, idx0, semi0).wait()
    gather(idx0, rows0, semg0).start()

    def outer(i, carry):
        c0 = 2 * i
        c1 = 2 * i + 1
        cn0 = jnp.minimum(c0 + 2, _NCHUNK - 1)
        cn1 = jnp.minimum(c1 + 2, _NCHUNK - 1)

        idx_cp(c1, idx1, semi1).wait()
        gather(idx1, rows1, semg1).start()
        gather(idx0, rows0, semg0).wait()
        denv0 = count(idx0)
        idx_cp(cn0, idx0, semi0).start()
        @pl.when(i > 0)
        def _():
            out_cp(c0 - 2, outb0, semo0).wait()
        process(c0, denv0, rows0, outb0, semo0)

        idx_cp(cn0, idx0, semi0).wait()
        gather(idx0, rows0, semg0).start()
        gather(idx1, rows1, semg1).wait()
        denv1 = count(idx1)
        idx_cp(cn1, idx1, semi1).start()
        @pl.when(i > 0)
        def _():
            out_cp(c1 - 2, outb1, semo1).wait()
        process(c1, denv1, rows1, outb1, semo1)
        return carry

    lax.fori_loop(0, _NCHUNK // 2, outer, 0)
    gather(idx0, rows0, semg0).wait()
    idx_cp(_NCHUNK - 1, idx1, semi1).wait()
    out_cp(_NCHUNK - 2, outb0, semo0).wait()
    out_cp(_NCHUNK - 1, outb1, semo1).wait()


def kernel(text_ids, table):
    ids_lm = (text_ids.reshape(_B // _R, _R, _S)
              .transpose(0, 2, 1)
              .reshape(-1))
    return _pool_kernel(ids_lm, table)


# speedup vs baseline: 2.7896x; 1.0009x over previous
import functools

import jax
import jax.numpy as jnp
from jax import lax
from jax.experimental import pallas as pl
from jax.experimental.pallas import tpu as pltpu
from jax.experimental.pallas import tpu_sc as plsc

_B = 16384
_S = 50
_D = 64
_NW = 32
_RPW = _B // _NW
_R = 16
_NCHUNK = _RPW // _R
_RS = _R * _S


@functools.partial(
    pl.kernel,
    mesh=plsc.VectorSubcoreMesh(core_axis_name="c", subcore_axis_name="s"),
    out_type=jax.ShapeDtypeStruct((_B, _D), jnp.float32),
    compiler_params=pltpu.CompilerParams(use_tc_tiling_on_sc=False),
    scratch_types=[
        pltpu.VMEM((_RS,), jnp.int32),
        pltpu.VMEM((_RS,), jnp.int32),
        pltpu.VMEM((_RS, _D), jnp.float32),
        pltpu.VMEM((_RS, _D), jnp.float32),
        pltpu.VMEM((_R, _D), jnp.float32),
        pltpu.VMEM((_R, _D), jnp.float32),
        pltpu.SemaphoreType.DMA,
        pltpu.SemaphoreType.DMA,
        pltpu.SemaphoreType.DMA,
        pltpu.SemaphoreType.DMA,
        pltpu.SemaphoreType.DMA,
        pltpu.SemaphoreType.DMA,
    ],
)
def _pool_kernel(idsm, table, out, idx0, idx1, rows0, rows1, outb0, outb1,
                 semi0, semi1, semg0, semg1, semo0, semo1):
    cid = lax.axis_index("c")
    sid = lax.axis_index("s")
    wid = sid * 2 + cid
    wbase = wid * _RPW

    ones = jnp.ones((16,), jnp.float32)
    zeros = jnp.zeros((16,), jnp.float32)

    def idx_cp(ci, idxref, sem):
        return pltpu.make_async_copy(
            idsm.at[pl.ds((wbase + ci * _R) * _S, _RS)], idxref, sem)

    def gather(idxref, rowsref, sem):
        return pltpu.make_async_copy(table.at[idxref], rowsref, sem)

    def out_cp(ci, outref, sem):
        return pltpu.make_async_copy(
            outref, out.at[pl.ds(wbase + ci * _R, _R)], sem)

    def count(idxref):
        def cnt_body(l, cv):
            return cv + jnp.where(idxref[pl.ds(l * 16, 16)] != 0, ones, zeros)

        cntv = lax.fori_loop(0, _S, cnt_body, zeros)
        return jnp.maximum(cntv, 1.0)

    def process(ci, denv, rowsref, outref, semo):
        for g in range(_R // 4):
            @plsc.parallel_loop(0, _S, carry=(zeros,) * 16)
            def acc_body(l, accs, g=g):
                res = []
                for rr in range(4):
                    o = l * _R + g * 4 + rr
                    for c in range(4):
                        res.append(accs[rr * 4 + c]
                                   + rowsref[o, pl.ds(c * 16, 16)])
                return tuple(res)

            accs = acc_body
            for rr in range(4):
                r = g * 4 + rr
                denom = lax.gather(
                    denv, jnp.full((16, 1), r, jnp.int32),
                    lax.GatherDimensionNumbers(
                        offset_dims=(), collapsed_slice_dims=(0,),
                        start_index_map=(0,)),
                    (1,), mode=lax.GatherScatterMode.PROMISE_IN_BOUNDS)
                for c in range(4):
                    outref[r, pl.ds(c * 16, 16)] = accs[rr * 4 + c] / denom

        out_cp(ci, outref, semo).start()

    idx_cp(0, idx0, semi0).start()
    idx_cp(1, idx1, semi1).start()
    idx_cp(0, idx0, semi0).wait()
    gather(idx0, rows0, semg0).start()

    def outer(i, carry):
        c0 = 2 * i
        c1 = 2 * i + 1
        cn0 = jnp.minimum(c0 + 2, _NCHUNK - 1)
        cn1 = jnp.minimum(c1 + 2, _NCHUNK - 1)

        idx_cp(c1, idx1, semi1).wait()
        gather(idx1, rows1, semg1).start()
        gather(idx0, rows0, semg0).wait()
        denv0 = count(idx0)
        idx_cp(cn0, idx0, semi0).start()
        @pl.when(i > 0)
        def _():
            out_cp(c0 - 2, outb0, semo0).wait()
        process(c0, denv0, rows0, outb0, semo0)

        idx_cp(cn0, idx0, semi0).wait()
        @pl.when(c0 + 2 < _NCHUNK)
        def _():
            gather(idx0, rows0, semg0).start()
        gather(idx1, rows1, semg1).wait()
        denv1 = count(idx1)
        idx_cp(cn1, idx1, semi1).start()
        @pl.when(i > 0)
        def _():
            out_cp(c1 - 2, outb1, semo1).wait()
        process(c1, denv1, rows1, outb1, semo1)
        return carry

    lax.fori_loop(0, _NCHUNK // 2, outer, 0)
    idx_cp(_NCHUNK - 1, idx1, semi1).wait()
    out_cp(_NCHUNK - 2, outb0, semo0).wait()
    out_cp(_NCHUNK - 1, outb1, semo1).wait()


def kernel(text_ids, table):
    ids_lm = (text_ids.reshape(_B // _R, _R, _S)
              .transpose(0, 2, 1)
              .reshape(-1))
    return _pool_kernel(ids_lm, table)
